# baseline (speedup 1.0000x reference)
import functools
import numpy as np
import jax
import jax.numpy as jnp
from jax.experimental import pallas as pl

D_MODEL = 256
N_HEADS = 8
N_LEVELS = 4
N_POINTS = 4
D_HEAD = 32
_SPATIAL = [(64, 64), (32, 32), (16, 16), (8, 8)]
LEN_IN = sum(h * w for h, w in _SPATIAL)
BATCH = 4
LQ = LEN_IN
QB = 680
NQB = LQ // QB

_j = np.arange(256)
_lj = (_j // 8) % 4
_xyj = _j % 2
_W_of_l = np.array([w for (_, w) in _SPATIAL], np.float32)
_H_of_l = np.array([h for (h, _) in _SPATIAL], np.float32)
_starts = np.concatenate([[0], np.cumsum([h * w for h, w in _SPATIAL])[:-1]]).astype(np.int64)

INV_NORM = np.where(_xyj == 0, 1.0 / _W_of_l[_lj], 1.0 / _H_of_l[_lj]).astype(np.float32)[None, :]

SREF = (( _lj * 2 + _xyj)[None, :] == np.arange(8)[:, None]).astype(np.float32)

SX = np.zeros((256, 128), np.float32)
SY = np.zeros((256, 128), np.float32)
for _jj in range(256):
    if _jj % 2 == 0:
        SX[_jj, _jj // 2] = 1.0
    else:
        SY[_jj, _jj // 2] = 1.0

_j2 = np.arange(128)
_l2 = (_j2 // 4) % 4
_h2 = _j2 // 16
WROW = _W_of_l[_l2][None, :]
HROW = _H_of_l[_l2][None, :]
WROW_I = _W_of_l[_l2].astype(np.int32)[None, :]
WM1 = (_W_of_l[_l2] - 1).astype(np.float32)[None, :]
HM1 = (_H_of_l[_l2] - 1).astype(np.float32)[None, :]
WM1_I = (_W_of_l[_l2] - 1).astype(np.int32)[None, :]
HM1_I = (_H_of_l[_l2] - 1).astype(np.int32)[None, :]
START_ROW = _starts[_l2].astype(np.int32)[None, :]
H_ROW = _h2.astype(np.int32)[None, :]

G = (( _j2 // 16)[None, :] == np.arange(8)[:, None]).astype(np.float32).T
GT = G.T

CROWS_F = np.concatenate([WROW, HROW, WM1, HM1], axis=0)
CROWS_I = np.concatenate([WROW_I, WM1_I, HM1_I, START_ROW, H_ROW], axis=0)


def _kernel_a(q_ref, ref8_ref, x_ref, woff_ref, boff_ref, wattn_ref, battn_ref,
              wval_ref, bval_ref, sref_ref, invn_ref, g_ref, gt_ref, sx_ref,
              sy_ref, cf_ref, ci_ref, val_ref, sloc_ref, idx_ref, w_ref):
    b = pl.program_id(0)
    q = q_ref[0]
    x = x_ref[0]

    val_ref[0] = jnp.dot(x, wval_ref[...], preferred_element_type=jnp.float32, precision=jax.lax.Precision.HIGHEST) + bval_ref[...]

    off = jnp.dot(q, woff_ref[...], preferred_element_type=jnp.float32, precision=jax.lax.Precision.HIGHEST) + boff_ref[...]
    refc = jnp.dot(ref8_ref[0], sref_ref[...], preferred_element_type=jnp.float32, precision=jax.lax.Precision.HIGHEST)
    sloc = refc + off * invn_ref[...]
    sloc_ref[0] = sloc

    logits = jnp.dot(q, wattn_ref[...], preferred_element_type=jnp.float32, precision=jax.lax.Precision.HIGHEST) + battn_ref[...]
    e = jnp.exp(logits)
    denom = jnp.dot(jnp.dot(e, g_ref[...], preferred_element_type=jnp.float32, precision=jax.lax.Precision.HIGHEST),
                    gt_ref[...], preferred_element_type=jnp.float32, precision=jax.lax.Precision.HIGHEST)
    attn = e / denom

    X = jnp.dot(sloc, sx_ref[...], preferred_element_type=jnp.float32, precision=jax.lax.Precision.HIGHEST)
    Y = jnp.dot(sloc, sy_ref[...], preferred_element_type=jnp.float32, precision=jax.lax.Precision.HIGHEST)
    wrow = cf_ref[0:1]
    hrow = cf_ref[1:2]
    wm1 = cf_ref[2:3]
    hm1 = cf_ref[3:4]
    xp = X * wrow - 0.5
    yp = Y * hrow - 0.5
    x0 = jnp.floor(xp)
    y0 = jnp.floor(yp)
    fx = xp - x0
    fy = yp - y0
    wx0 = 1.0 - fx
    wy0 = 1.0 - fy

    vx0 = ((x0 >= 0.0) & (x0 <= wm1)).astype(jnp.float32)
    vx1 = ((x0 + 1.0 >= 0.0) & (x0 + 1.0 <= wm1)).astype(jnp.float32)
    vy0 = ((y0 >= 0.0) & (y0 <= hm1)).astype(jnp.float32)
    vy1 = ((y0 + 1.0 >= 0.0) & (y0 + 1.0 <= hm1)).astype(jnp.float32)

    ix0 = x0.astype(jnp.int32)
    iy0 = y0.astype(jnp.int32)
    zero = jnp.zeros_like(ix0)
    wm1i = ci_ref[1:2]
    hm1i = ci_ref[2:3]
    cx0 = jnp.clip(ix0, zero, wm1i)
    cx1 = jnp.clip(ix0 + 1, zero, wm1i)
    cy0 = jnp.clip(iy0, zero, hm1i)
    cy1 = jnp.clip(iy0 + 1, zero, hm1i)

    base = b * (LQ * N_HEADS) + ci_ref[4:5]
    st = ci_ref[3:4]
    wi = ci_ref[0:1]

    def gidx(cx, cy):
        return (st + cy * wi + cx) * N_HEADS + base

    idx_ref[0, 0] = gidx(cx0, cy0)
    idx_ref[0, 1] = gidx(cx1, cy0)
    idx_ref[0, 2] = gidx(cx0, cy1)
    idx_ref[0, 3] = gidx(cx1, cy1)

    w_ref[0, 0] = wx0 * wy0 * vx0 * vy0 * attn
    w_ref[0, 1] = fx * wy0 * vx1 * vy0 * attn
    w_ref[0, 2] = wx0 * fy * vx0 * vy1 * attn
    w_ref[0, 3] = fx * fy * vx1 * vy1 * attn


def _kernel_c(acc_ref, wout_ref, bout_ref, out_ref):
    out_ref[0] = jnp.dot(acc_ref[0], wout_ref[...], preferred_element_type=jnp.float32, precision=jax.lax.Precision.HIGHEST) + bout_ref[...]


@jax.jit
def kernel(query, reference_points, input_flatten, input_spatial_shapes,
           input_level_start_index, W_off, b_off, W_attn, b_attn, W_val, b_val,
           W_out, b_out):
    del input_spatial_shapes, input_level_start_index
    ref8 = reference_points.reshape(BATCH, LQ, 8)

    full = lambda shp: pl.BlockSpec(shp, lambda b, qb: (0,) * len(shp))
    val, sloc, idx, w = pl.pallas_call(
        _kernel_a,
        grid=(BATCH, NQB),
        in_specs=[
            pl.BlockSpec((1, QB, 256), lambda b, qb: (b, qb, 0)),
            pl.BlockSpec((1, QB, 8), lambda b, qb: (b, qb, 0)),
            pl.BlockSpec((1, QB, 256), lambda b, qb: (b, qb, 0)),
            full((256, 256)), full((1, 256)),
            full((256, 128)), full((1, 128)),
            full((256, 256)), full((1, 256)),
            full((8, 256)), full((1, 256)), full((128, 8)), full((8, 128)),
            full((256, 128)), full((256, 128)), full((4, 128)), full((5, 128)),
        ],
        out_specs=[
            pl.BlockSpec((1, QB, 256), lambda b, qb: (b, qb, 0)),
            pl.BlockSpec((1, QB, 256), lambda b, qb: (b, qb, 0)),
            pl.BlockSpec((1, 4, QB, 128), lambda b, qb: (b, 0, qb, 0)),
            pl.BlockSpec((1, 4, QB, 128), lambda b, qb: (b, 0, qb, 0)),
        ],
        out_shape=[
            jax.ShapeDtypeStruct((BATCH, LEN_IN, 256), jnp.float32),
            jax.ShapeDtypeStruct((BATCH, LQ, 256), jnp.float32),
            jax.ShapeDtypeStruct((BATCH, 4, LQ, 128), jnp.int32),
            jax.ShapeDtypeStruct((BATCH, 4, LQ, 128), jnp.float32),
        ],
    )(query, ref8, input_flatten,
      W_off, b_off.reshape(1, 256), W_attn, b_attn.reshape(1, 128),
      W_val, b_val.reshape(1, 256),
      jnp.asarray(SREF), jnp.asarray(INV_NORM), jnp.asarray(G), jnp.asarray(GT),
      jnp.asarray(SX), jnp.asarray(SY), jnp.asarray(CROWS_F), jnp.asarray(CROWS_I))

    idx_r = idx.reshape(BATCH, 4, LQ, 8, 16).transpose(0, 2, 3, 1, 4).reshape(-1, 64)
    w_r = w.reshape(BATCH, 4, LQ, 8, 16).transpose(0, 2, 3, 1, 4).reshape(-1, 64)

    table = val.reshape(BATCH * LEN_IN * N_HEADS, D_HEAD)
    rows = jnp.take(table, idx_r, axis=0)
    acc = (rows * w_r[..., None]).sum(axis=1)
    acc = acc.reshape(BATCH, LQ, D_MODEL)

    out = pl.pallas_call(
        _kernel_c,
        grid=(BATCH, NQB),
        in_specs=[
            pl.BlockSpec((1, QB, 256), lambda b, qb: (b, qb, 0)),
            pl.BlockSpec((256, 256), lambda b, qb: (0, 0)),
            pl.BlockSpec((1, 256), lambda b, qb: (0, 0)),
        ],
        out_specs=pl.BlockSpec((1, QB, 256), lambda b, qb: (b, qb, 0)),
        out_shape=jax.ShapeDtypeStruct((BATCH, LQ, 256), jnp.float32),
    )(acc, W_out, b_out.reshape(1, 256))

    sampling_locations = sloc.reshape(BATCH, LQ, N_HEADS, N_LEVELS, N_POINTS, 2)
    return (out, sampling_locations)


# speedup vs baseline: 22.8760x; 22.8760x over previous
import functools
import numpy as np
import jax
import jax.numpy as jnp
from jax import lax
from jax.experimental import pallas as pl
from jax.experimental.pallas import tpu as pltpu
from jax.experimental.pallas import tpu_sc as plsc

D_MODEL = 256
N_HEADS = 8
N_LEVELS = 4
N_POINTS = 4
D_HEAD = 32
_SPATIAL = [(64, 64), (32, 32), (16, 16), (8, 8)]
LEN_IN = sum(h * w for h, w in _SPATIAL)
BATCH = 4
LQ = LEN_IN
QB = 680
NQB = LQ // QB

_j = np.arange(256)
_lj = (_j // 8) % 4
_xyj = _j % 2
_W_of_l = np.array([w for (_, w) in _SPATIAL], np.float32)
_H_of_l = np.array([h for (h, _) in _SPATIAL], np.float32)
_starts = np.concatenate([[0], np.cumsum([h * w for h, w in _SPATIAL])[:-1]]).astype(np.int64)

INV_NORM = np.where(_xyj == 0, 1.0 / _W_of_l[_lj], 1.0 / _H_of_l[_lj]).astype(np.float32)[None, :]

SREF = (( _lj * 2 + _xyj)[None, :] == np.arange(8)[:, None]).astype(np.float32)

SX = np.zeros((256, 128), np.float32)
SY = np.zeros((256, 128), np.float32)
for _jj in range(256):
    if _jj % 2 == 0:
        SX[_jj, _jj // 2] = 1.0
    else:
        SY[_jj, _jj // 2] = 1.0

_j2 = np.arange(128)
_l2 = (_j2 // 4) % 4
_h2 = _j2 // 16
WROW = _W_of_l[_l2][None, :]
HROW = _H_of_l[_l2][None, :]
WROW_I = _W_of_l[_l2].astype(np.int32)[None, :]
WM1 = (_W_of_l[_l2] - 1).astype(np.float32)[None, :]
HM1 = (_H_of_l[_l2] - 1).astype(np.float32)[None, :]
WM1_I = (_W_of_l[_l2] - 1).astype(np.int32)[None, :]
HM1_I = (_H_of_l[_l2] - 1).astype(np.int32)[None, :]
START_ROW = _starts[_l2].astype(np.int32)[None, :]
H_ROW = _h2.astype(np.int32)[None, :]

G = (( _j2 // 16)[None, :] == np.arange(8)[:, None]).astype(np.float32).T
GT = G.T

CROWS_F = np.concatenate([WROW, HROW, WM1, HM1], axis=0)
CROWS_I = np.concatenate([WROW_I, WM1_I, HM1_I, START_ROW, H_ROW], axis=0)


def _kernel_a(q_ref, ref8_ref, x_ref, woff_ref, boff_ref, wattn_ref, battn_ref,
              wval_ref, bval_ref, sref_ref, invn_ref, g_ref, gt_ref, sx_ref,
              sy_ref, cf_ref, ci_ref, val_ref, sloc_ref, idx_ref, w_ref):
    b = pl.program_id(0)
    q = q_ref[0]
    x = x_ref[0]

    val_ref[0] = jnp.dot(x, wval_ref[...], preferred_element_type=jnp.float32, precision=jax.lax.Precision.HIGHEST) + bval_ref[...]

    off = jnp.dot(q, woff_ref[...], preferred_element_type=jnp.float32, precision=jax.lax.Precision.HIGHEST) + boff_ref[...]
    refc = jnp.dot(ref8_ref[0], sref_ref[...], preferred_element_type=jnp.float32, precision=jax.lax.Precision.HIGHEST)
    sloc = refc + off * invn_ref[...]
    sloc_ref[0] = sloc

    logits = jnp.dot(q, wattn_ref[...], preferred_element_type=jnp.float32, precision=jax.lax.Precision.HIGHEST) + battn_ref[...]
    e = jnp.exp(logits)
    denom = jnp.dot(jnp.dot(e, g_ref[...], preferred_element_type=jnp.float32, precision=jax.lax.Precision.HIGHEST),
                    gt_ref[...], preferred_element_type=jnp.float32, precision=jax.lax.Precision.HIGHEST)
    attn = e / denom

    X = jnp.dot(sloc, sx_ref[...], preferred_element_type=jnp.float32, precision=jax.lax.Precision.HIGHEST)
    Y = jnp.dot(sloc, sy_ref[...], preferred_element_type=jnp.float32, precision=jax.lax.Precision.HIGHEST)
    wrow = cf_ref[0:1]
    hrow = cf_ref[1:2]
    wm1 = cf_ref[2:3]
    hm1 = cf_ref[3:4]
    xp = X * wrow - 0.5
    yp = Y * hrow - 0.5
    x0 = jnp.floor(xp)
    y0 = jnp.floor(yp)
    fx = xp - x0
    fy = yp - y0
    wx0 = 1.0 - fx
    wy0 = 1.0 - fy

    vx0 = ((x0 >= 0.0) & (x0 <= wm1)).astype(jnp.float32)
    vx1 = ((x0 + 1.0 >= 0.0) & (x0 + 1.0 <= wm1)).astype(jnp.float32)
    vy0 = ((y0 >= 0.0) & (y0 <= hm1)).astype(jnp.float32)
    vy1 = ((y0 + 1.0 >= 0.0) & (y0 + 1.0 <= hm1)).astype(jnp.float32)

    ix0 = x0.astype(jnp.int32)
    iy0 = y0.astype(jnp.int32)
    zero = jnp.zeros_like(ix0)
    wm1i = ci_ref[1:2]
    hm1i = ci_ref[2:3]
    cx0 = jnp.clip(ix0, zero, wm1i)
    cx1 = jnp.clip(ix0 + 1, zero, wm1i)
    cy0 = jnp.clip(iy0, zero, hm1i)
    cy1 = jnp.clip(iy0 + 1, zero, hm1i)

    base = b * (LQ * N_HEADS) + ci_ref[4:5]
    st = ci_ref[3:4]
    wi = ci_ref[0:1]

    def gidx(cx, cy):
        return (st + cy * wi + cx) * N_HEADS + base

    idx_ref[0, 0] = gidx(cx0, cy0)
    idx_ref[0, 1] = gidx(cx1, cy0)
    idx_ref[0, 2] = gidx(cx0, cy1)
    idx_ref[0, 3] = gidx(cx1, cy1)

    w_ref[0, 0] = wx0 * wy0 * vx0 * vy0 * attn
    w_ref[0, 1] = fx * wy0 * vx1 * vy0 * attn
    w_ref[0, 2] = wx0 * fy * vx0 * vy1 * attn
    w_ref[0, 3] = fx * fy * vx1 * vy1 * attn


N_GATHER = BATCH * LQ * N_HEADS * 64
N_WORKERS = 32
ROWS_PER_W = N_GATHER // N_WORKERS
CHUNK = 2048
N_CHUNKS = ROWS_PER_W // CHUNK


def _sc_gather(table_hbm, idx_hbm, out_hbm, idx_v, rows_v, sem):
    info = plsc.get_sparse_core_info()
    nc = info.num_cores
    wid = lax.axis_index("s") * nc + lax.axis_index("c")
    wbase = wid * ROWS_PER_W

    def body(i, _):
        base = pl.multiple_of(wbase + i * CHUNK, CHUNK)
        pltpu.sync_copy(idx_hbm.at[pl.ds(base, CHUNK)], idx_v)
        pltpu.async_copy(table_hbm.at[idx_v], rows_v, sem).wait()
        pltpu.sync_copy(rows_v, out_hbm.at[pl.ds(base, CHUNK)])
        return ()

    lax.fori_loop(0, N_CHUNKS, body, ())


_sc_gather_call = functools.partial(
    pl.kernel,
    mesh=plsc.VectorSubcoreMesh(core_axis_name="c", subcore_axis_name="s"),
    compiler_params=pltpu.CompilerParams(use_tc_tiling_on_sc=False),
    out_type=jax.ShapeDtypeStruct((N_GATHER, D_HEAD), jnp.float32),
    scratch_types=[
        pltpu.VMEM((CHUNK,), jnp.int32),
        pltpu.VMEM((CHUNK, D_HEAD), jnp.float32),
        pltpu.SemaphoreType.DMA,
    ],
)(_sc_gather)


RB = 544
NRB = (BATCH * LQ * N_HEADS) // RB


def _kernel_ws(rows_ref, w_ref, acc_ref):
    rows = rows_ref[...]
    w = w_ref[...]
    acc_ref[...] = (rows * w[..., None]).sum(axis=1)


def _kernel_c(acc_ref, wout_ref, bout_ref, out_ref):
    out_ref[0] = jnp.dot(acc_ref[0], wout_ref[...], preferred_element_type=jnp.float32, precision=jax.lax.Precision.HIGHEST) + bout_ref[...]


@jax.jit
def kernel(query, reference_points, input_flatten, input_spatial_shapes,
           input_level_start_index, W_off, b_off, W_attn, b_attn, W_val, b_val,
           W_out, b_out):
    del input_spatial_shapes, input_level_start_index
    ref8 = reference_points.reshape(BATCH, LQ, 8)

    full = lambda shp: pl.BlockSpec(shp, lambda b, qb: (0,) * len(shp))
    val, sloc, idx, w = pl.pallas_call(
        _kernel_a,
        grid=(BATCH, NQB),
        in_specs=[
            pl.BlockSpec((1, QB, 256), lambda b, qb: (b, qb, 0)),
            pl.BlockSpec((1, QB, 8), lambda b, qb: (b, qb, 0)),
            pl.BlockSpec((1, QB, 256), lambda b, qb: (b, qb, 0)),
            full((256, 256)), full((1, 256)),
            full((256, 128)), full((1, 128)),
            full((256, 256)), full((1, 256)),
            full((8, 256)), full((1, 256)), full((128, 8)), full((8, 128)),
            full((256, 128)), full((256, 128)), full((4, 128)), full((5, 128)),
        ],
        out_specs=[
            pl.BlockSpec((1, QB, 256), lambda b, qb: (b, qb, 0)),
            pl.BlockSpec((1, QB, 256), lambda b, qb: (b, qb, 0)),
            pl.BlockSpec((1, 4, QB, 128), lambda b, qb: (b, 0, qb, 0)),
            pl.BlockSpec((1, 4, QB, 128), lambda b, qb: (b, 0, qb, 0)),
        ],
        out_shape=[
            jax.ShapeDtypeStruct((BATCH, LEN_IN, 256), jnp.float32),
            jax.ShapeDtypeStruct((BATCH, LQ, 256), jnp.float32),
            jax.ShapeDtypeStruct((BATCH, 4, LQ, 128), jnp.int32),
            jax.ShapeDtypeStruct((BATCH, 4, LQ, 128), jnp.float32),
        ],
    )(query, ref8, input_flatten,
      W_off, b_off.reshape(1, 256), W_attn, b_attn.reshape(1, 128),
      W_val, b_val.reshape(1, 256),
      jnp.asarray(SREF), jnp.asarray(INV_NORM), jnp.asarray(G), jnp.asarray(GT),
      jnp.asarray(SX), jnp.asarray(SY), jnp.asarray(CROWS_F), jnp.asarray(CROWS_I))

    idx_r = idx.reshape(BATCH, 4, LQ, 8, 16).transpose(0, 2, 3, 1, 4).reshape(-1, 64)
    w_r = w.reshape(BATCH, 4, LQ, 8, 16).transpose(0, 2, 3, 1, 4).reshape(-1, 64)

    table = val.reshape(BATCH * LEN_IN * N_HEADS, D_HEAD)
    rows = _sc_gather_call(table, idx_r.reshape(-1))
    rows = rows.reshape(-1, 64, D_HEAD)

    acc = pl.pallas_call(
        _kernel_ws,
        grid=(NRB,),
        in_specs=[
            pl.BlockSpec((RB, 64, D_HEAD), lambda i: (i, 0, 0)),
            pl.BlockSpec((RB, 64), lambda i: (i, 0)),
        ],
        out_specs=pl.BlockSpec((RB, D_HEAD), lambda i: (i, 0)),
        out_shape=jax.ShapeDtypeStruct((BATCH * LQ * N_HEADS, D_HEAD), jnp.float32),
    )(rows, w_r)
    acc = acc.reshape(BATCH, LQ, D_MODEL)

    out = pl.pallas_call(
        _kernel_c,
        grid=(BATCH, NQB),
        in_specs=[
            pl.BlockSpec((1, QB, 256), lambda b, qb: (b, qb, 0)),
            pl.BlockSpec((256, 256), lambda b, qb: (0, 0)),
            pl.BlockSpec((1, 256), lambda b, qb: (0, 0)),
        ],
        out_specs=pl.BlockSpec((1, QB, 256), lambda b, qb: (b, qb, 0)),
        out_shape=jax.ShapeDtypeStruct((BATCH, LQ, 256), jnp.float32),
    )(acc, W_out, b_out.reshape(1, 256))

    sampling_locations = sloc.reshape(BATCH, LQ, N_HEADS, N_LEVELS, N_POINTS, 2)
    return (out, sampling_locations)


# speedup vs baseline: 35.3465x; 1.5451x over previous
import functools
import numpy as np
import jax
import jax.numpy as jnp
from jax import lax
from jax.experimental import pallas as pl
from jax.experimental.pallas import tpu as pltpu
from jax.experimental.pallas import tpu_sc as plsc

D_MODEL = 256
N_HEADS = 8
N_LEVELS = 4
N_POINTS = 4
D_HEAD = 32
_SPATIAL = [(64, 64), (32, 32), (16, 16), (8, 8)]
LEN_IN = sum(h * w for h, w in _SPATIAL)
BATCH = 4
LQ = LEN_IN
QB = 680
NQB = LQ // QB

_j = np.arange(256)
_lj = (_j // 8) % 4
_xyj = _j % 2
_W_of_l = np.array([w for (_, w) in _SPATIAL], np.float32)
_H_of_l = np.array([h for (h, _) in _SPATIAL], np.float32)
_starts = np.concatenate([[0], np.cumsum([h * w for h, w in _SPATIAL])[:-1]]).astype(np.int64)

INV_NORM = np.where(_xyj == 0, 1.0 / _W_of_l[_lj], 1.0 / _H_of_l[_lj]).astype(np.float32)[None, :]

SREF = (( _lj * 2 + _xyj)[None, :] == np.arange(8)[:, None]).astype(np.float32)

SX = np.zeros((256, 128), np.float32)
SY = np.zeros((256, 128), np.float32)
for _jj in range(256):
    if _jj % 2 == 0:
        SX[_jj, _jj // 2] = 1.0
    else:
        SY[_jj, _jj // 2] = 1.0

_j2 = np.arange(128)
_l2 = (_j2 // 4) % 4
_h2 = _j2 // 16
WROW = _W_of_l[_l2][None, :]
HROW = _H_of_l[_l2][None, :]
WROW_I = _W_of_l[_l2].astype(np.int32)[None, :]
WM1 = (_W_of_l[_l2] - 1).astype(np.float32)[None, :]
HM1 = (_H_of_l[_l2] - 1).astype(np.float32)[None, :]
WM1_I = (_W_of_l[_l2] - 1).astype(np.int32)[None, :]
HM1_I = (_H_of_l[_l2] - 1).astype(np.int32)[None, :]
START_ROW = _starts[_l2].astype(np.int32)[None, :]
H_ROW = _h2.astype(np.int32)[None, :]

G = (( _j2 // 16)[None, :] == np.arange(8)[:, None]).astype(np.float32).T
GT = G.T

CROWS_F = np.concatenate([WROW, HROW, WM1, HM1], axis=0)
CROWS_I = np.concatenate([WROW_I, WM1_I, HM1_I, START_ROW, H_ROW], axis=0)


def _kernel_a(q_ref, ref8_ref, x_ref, woff_ref, boff_ref, wattn_ref, battn_ref,
              wval_ref, bval_ref, sref_ref, invn_ref, g_ref, gt_ref, sx_ref,
              sy_ref, cf_ref, ci_ref, val_ref, sloc_ref, idx_ref, w_ref):
    b = pl.program_id(0)
    q = q_ref[0]
    x = x_ref[0]

    val_ref[0] = jnp.dot(x, wval_ref[...], preferred_element_type=jnp.float32, precision=jax.lax.Precision.HIGHEST) + bval_ref[...]

    off = jnp.dot(q, woff_ref[...], preferred_element_type=jnp.float32, precision=jax.lax.Precision.HIGHEST) + boff_ref[...]
    refc = jnp.dot(ref8_ref[0], sref_ref[...], preferred_element_type=jnp.float32, precision=jax.lax.Precision.HIGHEST)
    sloc = refc + off * invn_ref[...]
    sloc_ref[0] = sloc

    logits = jnp.dot(q, wattn_ref[...], preferred_element_type=jnp.float32, precision=jax.lax.Precision.HIGHEST) + battn_ref[...]
    e = jnp.exp(logits)
    denom = jnp.dot(jnp.dot(e, g_ref[...], preferred_element_type=jnp.float32, precision=jax.lax.Precision.HIGHEST),
                    gt_ref[...], preferred_element_type=jnp.float32, precision=jax.lax.Precision.HIGHEST)
    attn = e / denom

    X = jnp.dot(sloc, sx_ref[...], preferred_element_type=jnp.float32, precision=jax.lax.Precision.HIGHEST)
    Y = jnp.dot(sloc, sy_ref[...], preferred_element_type=jnp.float32, precision=jax.lax.Precision.HIGHEST)
    wrow = cf_ref[0:1]
    hrow = cf_ref[1:2]
    wm1 = cf_ref[2:3]
    hm1 = cf_ref[3:4]
    xp = X * wrow - 0.5
    yp = Y * hrow - 0.5
    x0 = jnp.floor(xp)
    y0 = jnp.floor(yp)
    fx = xp - x0
    fy = yp - y0
    wx0 = 1.0 - fx
    wy0 = 1.0 - fy

    vx0 = ((x0 >= 0.0) & (x0 <= wm1)).astype(jnp.float32)
    vx1 = ((x0 + 1.0 >= 0.0) & (x0 + 1.0 <= wm1)).astype(jnp.float32)
    vy0 = ((y0 >= 0.0) & (y0 <= hm1)).astype(jnp.float32)
    vy1 = ((y0 + 1.0 >= 0.0) & (y0 + 1.0 <= hm1)).astype(jnp.float32)

    ix0 = x0.astype(jnp.int32)
    iy0 = y0.astype(jnp.int32)
    zero = jnp.zeros_like(ix0)
    wm1i = ci_ref[1:2]
    hm1i = ci_ref[2:3]
    cx0 = jnp.clip(ix0, zero, wm1i)
    cx1 = jnp.clip(ix0 + 1, zero, wm1i)
    cy0 = jnp.clip(iy0, zero, hm1i)
    cy1 = jnp.clip(iy0 + 1, zero, hm1i)

    base = b * (LQ * N_HEADS) + ci_ref[4:5]
    st = ci_ref[3:4]
    wi = ci_ref[0:1]

    def gidx(cx, cy):
        return (st + cy * wi + cx) * N_HEADS + base

    idx_ref[0, 0] = gidx(cx0, cy0)
    idx_ref[0, 1] = gidx(cx1, cy0)
    idx_ref[0, 2] = gidx(cx0, cy1)
    idx_ref[0, 3] = gidx(cx1, cy1)

    w_ref[0, 0] = wx0 * wy0 * vx0 * vy0 * attn
    w_ref[0, 1] = fx * wy0 * vx1 * vy0 * attn
    w_ref[0, 2] = wx0 * fy * vx0 * vy1 * attn
    w_ref[0, 3] = fx * fy * vx1 * vy1 * attn


N_GATHER = BATCH * LQ * N_HEADS * 64
N_WORKERS = 32
ROWS_PER_W = N_GATHER // N_WORKERS
CHUNK = 2048
N_CHUNKS = ROWS_PER_W // CHUNK


def _sc_gather(table_hbm, idx_hbm, out_hbm, idx_v, rows_v, sem):
    info = plsc.get_sparse_core_info()
    nc = info.num_cores
    wid = lax.axis_index("s") * nc + lax.axis_index("c")
    wbase = wid * ROWS_PER_W

    def body(i, _):
        base = pl.multiple_of(wbase + i * CHUNK, CHUNK)
        pltpu.sync_copy(idx_hbm.at[pl.ds(base, CHUNK)], idx_v)
        pltpu.async_copy(table_hbm.at[idx_v], rows_v, sem).wait()
        pltpu.sync_copy(rows_v, out_hbm.at[pl.ds(base, CHUNK)])
        return ()

    lax.fori_loop(0, N_CHUNKS, body, ())


_sc_gather_call = functools.partial(
    pl.kernel,
    mesh=plsc.VectorSubcoreMesh(core_axis_name="c", subcore_axis_name="s"),
    compiler_params=pltpu.CompilerParams(use_tc_tiling_on_sc=False),
    out_type=jax.ShapeDtypeStruct((N_GATHER, D_HEAD), jnp.float32),
    scratch_types=[
        pltpu.VMEM((CHUNK,), jnp.int32),
        pltpu.VMEM((CHUNK, D_HEAD), jnp.float32),
        pltpu.SemaphoreType.DMA,
    ],
)(_sc_gather)


RB = 512
NRB = (BATCH * LQ * N_HEADS) // RB


EXPAND = ((np.arange(2048)[None, :] // 32) == np.arange(64)[:, None]).astype(np.float32)


def _kernel_ws(rows_ref, w_ref, e_ref, acc_ref):
    t = rows_ref[...]
    wexp = jnp.dot(w_ref[...], e_ref[...],
                   preferred_element_type=jnp.float32,
                   precision=jax.lax.Precision.HIGHEST)
    t = t * wexp
    k = 2048
    while k > 32:
        k //= 2
        t = t[:, :k] + t[:, k:]
    acc_ref[...] = t


def _kernel_c(acc_ref, wout_ref, bout_ref, out_ref):
    out_ref[0] = jnp.dot(acc_ref[0], wout_ref[...], preferred_element_type=jnp.float32, precision=jax.lax.Precision.HIGHEST) + bout_ref[...]


@jax.jit
def kernel(query, reference_points, input_flatten, input_spatial_shapes,
           input_level_start_index, W_off, b_off, W_attn, b_attn, W_val, b_val,
           W_out, b_out):
    del input_spatial_shapes, input_level_start_index
    ref8 = reference_points.reshape(BATCH, LQ, 8)

    full = lambda shp: pl.BlockSpec(shp, lambda b, qb: (0,) * len(shp))
    val, sloc, idx, w = pl.pallas_call(
        _kernel_a,
        grid=(BATCH, NQB),
        in_specs=[
            pl.BlockSpec((1, QB, 256), lambda b, qb: (b, qb, 0)),
            pl.BlockSpec((1, QB, 8), lambda b, qb: (b, qb, 0)),
            pl.BlockSpec((1, QB, 256), lambda b, qb: (b, qb, 0)),
            full((256, 256)), full((1, 256)),
            full((256, 128)), full((1, 128)),
            full((256, 256)), full((1, 256)),
            full((8, 256)), full((1, 256)), full((128, 8)), full((8, 128)),
            full((256, 128)), full((256, 128)), full((4, 128)), full((5, 128)),
        ],
        out_specs=[
            pl.BlockSpec((1, QB, 256), lambda b, qb: (b, qb, 0)),
            pl.BlockSpec((1, QB, 256), lambda b, qb: (b, qb, 0)),
            pl.BlockSpec((1, 4, QB, 128), lambda b, qb: (b, 0, qb, 0)),
            pl.BlockSpec((1, 4, QB, 128), lambda b, qb: (b, 0, qb, 0)),
        ],
        out_shape=[
            jax.ShapeDtypeStruct((BATCH, LEN_IN, 256), jnp.float32),
            jax.ShapeDtypeStruct((BATCH, LQ, 256), jnp.float32),
            jax.ShapeDtypeStruct((BATCH, 4, LQ, 128), jnp.int32),
            jax.ShapeDtypeStruct((BATCH, 4, LQ, 128), jnp.float32),
        ],
    )(query, ref8, input_flatten,
      W_off, b_off.reshape(1, 256), W_attn, b_attn.reshape(1, 128),
      W_val, b_val.reshape(1, 256),
      jnp.asarray(SREF), jnp.asarray(INV_NORM), jnp.asarray(G), jnp.asarray(GT),
      jnp.asarray(SX), jnp.asarray(SY), jnp.asarray(CROWS_F), jnp.asarray(CROWS_I))

    idx_r = idx.reshape(BATCH, 4, LQ, 8, 16).transpose(0, 2, 3, 1, 4).reshape(-1, 64)
    w_r = w.reshape(BATCH, 4, LQ, 8, 16).transpose(0, 2, 3, 1, 4).reshape(-1, 64)

    table = val.reshape(BATCH * LEN_IN * N_HEADS, D_HEAD)
    rows = _sc_gather_call(table, idx_r.reshape(-1))
    rows2d = rows.reshape(-1, 64 * D_HEAD)

    acc = pl.pallas_call(
        _kernel_ws,
        grid=(NRB,),
        in_specs=[
            pl.BlockSpec((RB, 64 * D_HEAD), lambda i: (i, 0)),
            pl.BlockSpec((RB, 64), lambda i: (i, 0)),
            pl.BlockSpec((64, 2048), lambda i: (0, 0)),
        ],
        out_specs=pl.BlockSpec((RB, D_HEAD), lambda i: (i, 0)),
        out_shape=jax.ShapeDtypeStruct((BATCH * LQ * N_HEADS, D_HEAD), jnp.float32),
    )(rows2d, w_r, jnp.asarray(EXPAND))
    acc = acc.reshape(BATCH, LQ, D_MODEL)

    out = pl.pallas_call(
        _kernel_c,
        grid=(BATCH, NQB),
        in_specs=[
            pl.BlockSpec((1, QB, 256), lambda b, qb: (b, qb, 0)),
            pl.BlockSpec((256, 256), lambda b, qb: (0, 0)),
            pl.BlockSpec((1, 256), lambda b, qb: (0, 0)),
        ],
        out_specs=pl.BlockSpec((1, QB, 256), lambda b, qb: (b, qb, 0)),
        out_shape=jax.ShapeDtypeStruct((BATCH, LQ, 256), jnp.float32),
    )(acc, W_out, b_out.reshape(1, 256))

    sampling_locations = sloc.reshape(BATCH, LQ, N_HEADS, N_LEVELS, N_POINTS, 2)
    return (out, sampling_locations)


# speedup vs baseline: 36.2606x; 1.0259x over previous
import functools
import numpy as np
import jax
import jax.numpy as jnp
from jax import lax
from jax.experimental import pallas as pl
from jax.experimental.pallas import tpu as pltpu
from jax.experimental.pallas import tpu_sc as plsc

D_MODEL = 256
N_HEADS = 8
N_LEVELS = 4
N_POINTS = 4
D_HEAD = 32
_SPATIAL = [(64, 64), (32, 32), (16, 16), (8, 8)]
LEN_IN = sum(h * w for h, w in _SPATIAL)
BATCH = 4
LQ = LEN_IN
QB = 680
NQB = LQ // QB

_j = np.arange(256)
_lj = (_j // 8) % 4
_xyj = _j % 2
_W_of_l = np.array([w for (_, w) in _SPATIAL], np.float32)
_H_of_l = np.array([h for (h, _) in _SPATIAL], np.float32)
_starts = np.concatenate([[0], np.cumsum([h * w for h, w in _SPATIAL])[:-1]]).astype(np.int64)

INV_NORM = np.where(_xyj == 0, 1.0 / _W_of_l[_lj], 1.0 / _H_of_l[_lj]).astype(np.float32)[None, :]

SREF = (( _lj * 2 + _xyj)[None, :] == np.arange(8)[:, None]).astype(np.float32)

SX = np.zeros((256, 128), np.float32)
SY = np.zeros((256, 128), np.float32)
for _jj in range(256):
    if _jj % 2 == 0:
        SX[_jj, _jj // 2] = 1.0
    else:
        SY[_jj, _jj // 2] = 1.0

_j2 = np.arange(128)
_l2 = (_j2 // 4) % 4
_h2 = _j2 // 16
WROW = _W_of_l[_l2][None, :]
HROW = _H_of_l[_l2][None, :]
WROW_I = _W_of_l[_l2].astype(np.int32)[None, :]
WM1 = (_W_of_l[_l2] - 1).astype(np.float32)[None, :]
HM1 = (_H_of_l[_l2] - 1).astype(np.float32)[None, :]
WM1_I = (_W_of_l[_l2] - 1).astype(np.int32)[None, :]
HM1_I = (_H_of_l[_l2] - 1).astype(np.int32)[None, :]
START_ROW = _starts[_l2].astype(np.int32)[None, :]
H_ROW = _h2.astype(np.int32)[None, :]

G = (( _j2 // 16)[None, :] == np.arange(8)[:, None]).astype(np.float32).T
GT = G.T

CROWS_F = np.concatenate([WROW, HROW, WM1, HM1], axis=0)
CROWS_I = np.concatenate([WROW_I, WM1_I, HM1_I, START_ROW, H_ROW], axis=0)


def _kernel_a(q_ref, ref8_ref, x_ref, woff_ref, boff_ref, wattn_ref, battn_ref,
              wval_ref, bval_ref, sref_ref, invn_ref, g_ref, gt_ref, sx_ref,
              sy_ref, cf_ref, ci_ref, val_ref, sloc_ref, idx_ref, w_ref):
    b = pl.program_id(0)
    q = q_ref[0]
    x = x_ref[0]

    val_ref[0] = jnp.dot(x, wval_ref[...], preferred_element_type=jnp.float32, precision=jax.lax.Precision.HIGHEST) + bval_ref[...]

    off = jnp.dot(q, woff_ref[...], preferred_element_type=jnp.float32, precision=jax.lax.Precision.HIGHEST) + boff_ref[...]
    refc = jnp.dot(ref8_ref[0], sref_ref[...], preferred_element_type=jnp.float32, precision=jax.lax.Precision.HIGHEST)
    sloc = refc + off * invn_ref[...]
    sloc_ref[0] = sloc

    logits = jnp.dot(q, wattn_ref[...], preferred_element_type=jnp.float32, precision=jax.lax.Precision.HIGHEST) + battn_ref[...]
    e = jnp.exp(logits)
    denom = jnp.dot(jnp.dot(e, g_ref[...], preferred_element_type=jnp.float32, precision=jax.lax.Precision.HIGHEST),
                    gt_ref[...], preferred_element_type=jnp.float32, precision=jax.lax.Precision.HIGHEST)
    attn = e / denom

    X = jnp.dot(sloc, sx_ref[...], preferred_element_type=jnp.float32, precision=jax.lax.Precision.HIGHEST)
    Y = jnp.dot(sloc, sy_ref[...], preferred_element_type=jnp.float32, precision=jax.lax.Precision.HIGHEST)
    wrow = cf_ref[0:1]
    hrow = cf_ref[1:2]
    wm1 = cf_ref[2:3]
    hm1 = cf_ref[3:4]
    xp = X * wrow - 0.5
    yp = Y * hrow - 0.5
    x0 = jnp.floor(xp)
    y0 = jnp.floor(yp)
    fx = xp - x0
    fy = yp - y0
    wx0 = 1.0 - fx
    wy0 = 1.0 - fy

    vx0 = ((x0 >= 0.0) & (x0 <= wm1)).astype(jnp.float32)
    vx1 = ((x0 + 1.0 >= 0.0) & (x0 + 1.0 <= wm1)).astype(jnp.float32)
    vy0 = ((y0 >= 0.0) & (y0 <= hm1)).astype(jnp.float32)
    vy1 = ((y0 + 1.0 >= 0.0) & (y0 + 1.0 <= hm1)).astype(jnp.float32)

    ix0 = x0.astype(jnp.int32)
    iy0 = y0.astype(jnp.int32)
    zero = jnp.zeros_like(ix0)
    wm1i = ci_ref[1:2]
    hm1i = ci_ref[2:3]
    cx0 = jnp.clip(ix0, zero, wm1i)
    cx1 = jnp.clip(ix0 + 1, zero, wm1i)
    cy0 = jnp.clip(iy0, zero, hm1i)
    cy1 = jnp.clip(iy0 + 1, zero, hm1i)

    base = b * (LQ * N_HEADS) + ci_ref[4:5]
    st = ci_ref[3:4]
    wi = ci_ref[0:1]

    def gidx(cx, cy):
        return (st + cy * wi + cx) * N_HEADS + base

    idx_ref[0, 0] = gidx(cx0, cy0)
    idx_ref[0, 1] = gidx(cx1, cy0)
    idx_ref[0, 2] = gidx(cx0, cy1)
    idx_ref[0, 3] = gidx(cx1, cy1)

    w_ref[0, 0] = wx0 * wy0 * vx0 * vy0 * attn
    w_ref[0, 1] = fx * wy0 * vx1 * vy0 * attn
    w_ref[0, 2] = wx0 * fy * vx0 * vy1 * attn
    w_ref[0, 3] = fx * fy * vx1 * vy1 * attn


N_GATHER = BATCH * LQ * N_HEADS * 64
N_WORKERS = 32
ROWS_PER_W = N_GATHER // N_WORKERS
CHUNK = 1024
N_PAIRS = ROWS_PER_W // (2 * CHUNK)


def _sc_gather(table_hbm, idx_hbm, out_hbm, idx_v0, idx_v1, rows_v0, rows_v1,
               sg0, sg1, sw0, sw1):
    info = plsc.get_sparse_core_info()
    nc = info.num_cores
    wid = lax.axis_index("s") * nc + lax.axis_index("c")
    wbase = wid * ROWS_PER_W

    def body(i, _):
        base0 = pl.multiple_of(wbase + (2 * i) * CHUNK, CHUNK)
        base1 = pl.multiple_of(wbase + (2 * i + 1) * CHUNK, CHUNK)
        pltpu.sync_copy(idx_hbm.at[pl.ds(base0, CHUNK)], idx_v0)
        g0 = pltpu.async_copy(table_hbm.at[idx_v0], rows_v0, sg0)
        pltpu.sync_copy(idx_hbm.at[pl.ds(base1, CHUNK)], idx_v1)
        g1 = pltpu.async_copy(table_hbm.at[idx_v1], rows_v1, sg1)
        g0.wait()
        w0 = pltpu.async_copy(rows_v0, out_hbm.at[pl.ds(base0, CHUNK)], sw0)
        g1.wait()
        w1 = pltpu.async_copy(rows_v1, out_hbm.at[pl.ds(base1, CHUNK)], sw1)
        w0.wait()
        w1.wait()
        return ()

    lax.fori_loop(0, N_PAIRS, body, ())


_sc_gather_call = functools.partial(
    pl.kernel,
    mesh=plsc.VectorSubcoreMesh(core_axis_name="c", subcore_axis_name="s"),
    compiler_params=pltpu.CompilerParams(use_tc_tiling_on_sc=False),
    out_type=jax.ShapeDtypeStruct((N_GATHER, D_HEAD), jnp.float32),
    scratch_types=[
        pltpu.VMEM((CHUNK,), jnp.int32),
        pltpu.VMEM((CHUNK,), jnp.int32),
        pltpu.VMEM((CHUNK, D_HEAD), jnp.float32),
        pltpu.VMEM((CHUNK, D_HEAD), jnp.float32),
        pltpu.SemaphoreType.DMA,
        pltpu.SemaphoreType.DMA,
        pltpu.SemaphoreType.DMA,
        pltpu.SemaphoreType.DMA,
    ],
)(_sc_gather)


RB = 512
NRB = (BATCH * LQ * N_HEADS) // RB


EXPAND = ((np.arange(2048)[None, :] // 32) == np.arange(64)[:, None]).astype(np.float32)


def _kernel_ws(rows_ref, w_ref, e_ref, acc_ref):
    t = rows_ref[...]
    wexp = jnp.dot(w_ref[...], e_ref[...],
                   preferred_element_type=jnp.float32,
                   precision=jax.lax.Precision.HIGHEST)
    t = t * wexp
    k = 2048
    while k > 32:
        k //= 2
        t = t[:, :k] + t[:, k:]
    acc_ref[...] = t


def _kernel_c(acc_ref, wout_ref, bout_ref, out_ref):
    out_ref[0] = jnp.dot(acc_ref[0], wout_ref[...], preferred_element_type=jnp.float32, precision=jax.lax.Precision.HIGHEST) + bout_ref[...]


@jax.jit
def kernel(query, reference_points, input_flatten, input_spatial_shapes,
           input_level_start_index, W_off, b_off, W_attn, b_attn, W_val, b_val,
           W_out, b_out):
    del input_spatial_shapes, input_level_start_index
    ref8 = reference_points.reshape(BATCH, LQ, 8)

    full = lambda shp: pl.BlockSpec(shp, lambda b, qb: (0,) * len(shp))
    val, sloc, idx, w = pl.pallas_call(
        _kernel_a,
        grid=(BATCH, NQB),
        in_specs=[
            pl.BlockSpec((1, QB, 256), lambda b, qb: (b, qb, 0)),
            pl.BlockSpec((1, QB, 8), lambda b, qb: (b, qb, 0)),
            pl.BlockSpec((1, QB, 256), lambda b, qb: (b, qb, 0)),
            full((256, 256)), full((1, 256)),
            full((256, 128)), full((1, 128)),
            full((256, 256)), full((1, 256)),
            full((8, 256)), full((1, 256)), full((128, 8)), full((8, 128)),
            full((256, 128)), full((256, 128)), full((4, 128)), full((5, 128)),
        ],
        out_specs=[
            pl.BlockSpec((1, QB, 256), lambda b, qb: (b, qb, 0)),
            pl.BlockSpec((1, QB, 256), lambda b, qb: (b, qb, 0)),
            pl.BlockSpec((1, 4, QB, 128), lambda b, qb: (b, 0, qb, 0)),
            pl.BlockSpec((1, 4, QB, 128), lambda b, qb: (b, 0, qb, 0)),
        ],
        out_shape=[
            jax.ShapeDtypeStruct((BATCH, LEN_IN, 256), jnp.float32),
            jax.ShapeDtypeStruct((BATCH, LQ, 256), jnp.float32),
            jax.ShapeDtypeStruct((BATCH, 4, LQ, 128), jnp.int32),
            jax.ShapeDtypeStruct((BATCH, 4, LQ, 128), jnp.float32),
        ],
    )(query, ref8, input_flatten,
      W_off, b_off.reshape(1, 256), W_attn, b_attn.reshape(1, 128),
      W_val, b_val.reshape(1, 256),
      jnp.asarray(SREF), jnp.asarray(INV_NORM), jnp.asarray(G), jnp.asarray(GT),
      jnp.asarray(SX), jnp.asarray(SY), jnp.asarray(CROWS_F), jnp.asarray(CROWS_I))

    idx_r = idx.reshape(BATCH, 4, LQ, 8, 16).transpose(0, 2, 3, 1, 4).reshape(-1, 64)
    w_r = w.reshape(BATCH, 4, LQ, 8, 16).transpose(0, 2, 3, 1, 4).reshape(-1, 64)

    table = val.reshape(BATCH * LEN_IN * N_HEADS, D_HEAD)
    rows = _sc_gather_call(table, idx_r.reshape(-1))
    rows2d = rows.reshape(-1, 64 * D_HEAD)

    acc = pl.pallas_call(
        _kernel_ws,
        grid=(NRB,),
        in_specs=[
            pl.BlockSpec((RB, 64 * D_HEAD), lambda i: (i, 0)),
            pl.BlockSpec((RB, 64), lambda i: (i, 0)),
            pl.BlockSpec((64, 2048), lambda i: (0, 0)),
        ],
        out_specs=pl.BlockSpec((RB, D_HEAD), lambda i: (i, 0)),
        out_shape=jax.ShapeDtypeStruct((BATCH * LQ * N_HEADS, D_HEAD), jnp.float32),
    )(rows2d, w_r, jnp.asarray(EXPAND))
    acc = acc.reshape(BATCH, LQ, D_MODEL)

    out = pl.pallas_call(
        _kernel_c,
        grid=(BATCH, NQB),
        in_specs=[
            pl.BlockSpec((1, QB, 256), lambda b, qb: (b, qb, 0)),
            pl.BlockSpec((256, 256), lambda b, qb: (0, 0)),
            pl.BlockSpec((1, 256), lambda b, qb: (0, 0)),
        ],
        out_specs=pl.BlockSpec((1, QB, 256), lambda b, qb: (b, qb, 0)),
        out_shape=jax.ShapeDtypeStruct((BATCH, LQ, 256), jnp.float32),
    )(acc, W_out, b_out.reshape(1, 256))

    sampling_locations = sloc.reshape(BATCH, LQ, N_HEADS, N_LEVELS, N_POINTS, 2)
    return (out, sampling_locations)
